# Initial kernel scaffold; baseline (speedup 1.0000x reference)
import jax
import jax.numpy as jnp
from jax.experimental import pallas as pl


def kernel(x, Wqkv, bqkv, Wproj, bproj):
    raise NotImplementedError("write your pallas kernel here")


# speedup vs baseline: 1.9498x; 1.9498x over previous
import functools

import jax
import jax.numpy as jnp
from jax.experimental import pallas as pl
from jax.experimental.pallas import tpu as pltpu

DIM = 1024
INNER = 1024
HEADS = 16
DH = INNER // HEADS
L = 2048
QCHUNK = 512


def _matmul_bias_kernel(x_ref, w_ref, b_ref, o_ref):
    o_ref[...] = (
        jnp.dot(x_ref[...], w_ref[...], preferred_element_type=jnp.float32)
        + b_ref[...]
    )


def _matmul_bias(x2d, w, b, mblk):
    m, k = x2d.shape
    n = w.shape[1]
    return pl.pallas_call(
        _matmul_bias_kernel,
        grid=(m // mblk,),
        in_specs=[
            pl.BlockSpec((mblk, k), lambda i: (i, 0)),
            pl.BlockSpec((k, n), lambda i: (0, 0)),
            pl.BlockSpec((1, n), lambda i: (0, 0)),
        ],
        out_specs=pl.BlockSpec((mblk, n), lambda i: (i, 0)),
        out_shape=jax.ShapeDtypeStruct((m, n), jnp.float32),
    )(x2d, w, b.reshape(1, n))


def _attn_kernel(q_ref, k_ref, v_ref, o_ref, *, scale):
    q2 = q_ref[0]
    k2 = k_ref[0]
    v2 = v_ref[0]
    lane = jax.lax.broadcasted_iota(jnp.int32, (1, 2 * DH), 1)
    mask_a = (lane < DH).astype(jnp.float32)
    mask_b = 1.0 - mask_a
    out = None
    for mask in (mask_a, mask_b):
        s = jax.lax.dot_general(
            q2 * mask, k2, (((1,), (1,)), ((), ())),
            preferred_element_type=jnp.float32,
        ) * scale
        m = jnp.max(s, axis=-1, keepdims=True)
        p = jnp.exp(s - m)
        l = jnp.sum(p, axis=-1, keepdims=True)
        o = jnp.dot(p, v2, preferred_element_type=jnp.float32)
        o = o * (mask / l)
        out = o if out is None else out + o
    o_ref[0] = out


def _attention(qkv, batch):
    npair = HEADS // 2
    grid = (batch, npair, L // QCHUNK)
    scale = DH ** (-0.5)
    return pl.pallas_call(
        functools.partial(_attn_kernel, scale=scale),
        grid=grid,
        in_specs=[
            pl.BlockSpec((1, QCHUNK, 2 * DH), lambda b, j, g: (b, g, j)),
            pl.BlockSpec((1, L, 2 * DH), lambda b, j, g: (b, 0, npair + j)),
            pl.BlockSpec((1, L, 2 * DH), lambda b, j, g: (b, 0, 2 * npair + j)),
        ],
        out_specs=pl.BlockSpec((1, QCHUNK, 2 * DH), lambda b, j, g: (b, g, j)),
        out_shape=jax.ShapeDtypeStruct((batch, L, INNER), jnp.float32),
    )(qkv, qkv, qkv)


def kernel(x, Wqkv, bqkv, Wproj, bproj):
    b, l, d = x.shape
    qkv = _matmul_bias(x.reshape(b * l, d), Wqkv, bqkv, 512)
    attn = _attention(qkv.reshape(b, l, 3 * INNER), b)
    out = _matmul_bias(attn.reshape(b * l, INNER), Wproj, bproj, 512)
    return out.reshape(b, l, DIM)
